# baseline (speedup 1.0000x reference)
import jax
import jax.numpy as jnp
from jax.experimental import pallas as pl

N = 10000
G = 400


def _copy_kernel(x_ref, o_ref):
    o_ref[...] = x_ref[...]


def kernel(x, edge_index, batch, DDI_features, protein_mask, W1_root, W1_rel, b1, W2_root, W2_rel, b2, W3_root, W3_rel, b3, lin1_W, lin1_b, lin2_W, lin2_b, lin3_W, lin3_b):
    src = edge_index[0]
    dst = edge_index[1]

    def gconv(h, Wr, Wn, b):
        agg = jax.ops.segment_sum(h @ Wn, dst, num_segments=N, indices_are_sorted=False)
        return agg, h @ Wr + b

    def gconv_full(h, Wr, Wn, b):
        a, r = gconv(h, Wr, Wn, b)
        return jax.nn.relu(a[src if False else slice(None)] + r) if False else jax.nn.relu(a + r)

    def pool(h):
        mx = jax.ops.segment_max(h, batch, num_segments=G)
        sm = jax.ops.segment_sum(h, batch, num_segments=G)
        return jnp.concatenate([mx, sm], axis=1)

    def gc(h, Wr, Wn, b):
        agg = jax.ops.segment_sum(h[src], dst, num_segments=N)
        return agg @ Wn + h @ Wr + b

    h = jax.nn.relu(gc(x, W1_root, W1_rel, b1))
    x1 = pool(h)
    h = jax.nn.relu(gc(h, W2_root, W2_rel, b2))
    x2 = pool(h)
    h = jax.nn.relu(gc(h, W3_root, W3_rel, b3))
    x3 = pool(h)

    s = x1 + x2 + x3
    s = s.reshape(32, 25, 128)
    m = protein_mask.reshape(32, 1, 25).astype(jnp.float32)
    v = jnp.matmul(m, s).reshape(32, -1)
    v = jnp.concatenate([v, DDI_features], axis=1)
    v = jax.nn.relu(v @ lin1_W + lin1_b)
    v = jax.nn.relu(v @ lin2_W + lin2_b)
    out = v @ lin3_W + lin3_b
    out = pl.pallas_call(
        _copy_kernel,
        out_shape=jax.ShapeDtypeStruct(out.shape, out.dtype),
    )(out)
    return out


# speedup vs baseline: 6.9012x; 6.9012x over previous
import functools

import jax
import jax.numpy as jnp
from jax import lax
from jax.experimental import pallas as pl
from jax.experimental.pallas import tpu as pltpu
from jax.experimental.pallas import tpu_sc as plsc

N = 10000
E = 160000
G = 400
GP = 416
B = 32
P = 25
D_IN = 256
D_H = 128

NC = 2
NS = 16
NW = NC * NS
EPW = E // NW
WIN = 100
NWIN = EPW // WIN
CH = 32
SLAB = 624
TAIL = N - NS * SLAB
RB = 400

_F32 = jnp.float32
_NEG_INF = float("-inf")


def _offsets_body(batch_ref, off_ref):
    i = pl.program_id(0)
    b = batch_ref[...]
    j = lax.broadcasted_iota(jnp.int32, (RB, 512), 1)
    cnt = jnp.sum((b < j).astype(jnp.int32), axis=0, keepdims=True)

    @pl.when(i == 0)
    def _():
        off_ref[...] = jnp.zeros_like(off_ref)

    off_ref[...] += cnt


def _compute_offsets(batch):
    bcol = batch.reshape(N, 1)
    out = pl.pallas_call(
        _offsets_body,
        grid=(N // RB,),
        in_specs=[pl.BlockSpec((RB, 1), lambda i: (i, 0))],
        out_specs=pl.BlockSpec((1, 512), lambda i: (0, 0)),
        out_shape=jax.ShapeDtypeStruct((1, 512), jnp.int32),
    )(bcol)
    return out.reshape(512)


def _proj_body(h_ref, wn_ref, wr_ref, b_ref, m_ref, r_ref):
    h = h_ref[...]
    m_ref[...] = jnp.dot(h, wn_ref[...], preferred_element_type=_F32, precision=lax.Precision.HIGHEST)
    r_ref[...] = jnp.dot(h, wr_ref[...], preferred_element_type=_F32, precision=lax.Precision.HIGHEST) + b_ref[...]


def _proj(h, Wn, Wr, b):
    din = h.shape[1]
    return pl.pallas_call(
        _proj_body,
        grid=(N // RB,),
        in_specs=[
            pl.BlockSpec((RB, din), lambda i: (i, 0)),
            pl.BlockSpec((din, D_H), lambda i: (0, 0)),
            pl.BlockSpec((din, D_H), lambda i: (0, 0)),
            pl.BlockSpec((1, D_H), lambda i: (0, 0)),
        ],
        out_specs=[
            pl.BlockSpec((RB, D_H), lambda i: (i, 0)),
            pl.BlockSpec((RB, D_H), lambda i: (i, 0)),
        ],
        out_shape=[
            jax.ShapeDtypeStruct((N, D_H), _F32),
            jax.ShapeDtypeStruct((N, D_H), _F32),
        ],
    )(h, Wn, Wr, b.reshape(1, D_H))


def _relu_proj_body(p0_ref, p1_ref, r_ref, wn_ref, wr_ref, b_ref,
                    h_ref, m2_ref, r2_ref):
    h = jnp.maximum(p0_ref[...] + p1_ref[...] + r_ref[...], 0.0)
    h_ref[...] = h
    m2_ref[...] = jnp.dot(h, wn_ref[...], preferred_element_type=_F32, precision=lax.Precision.HIGHEST)
    r2_ref[...] = jnp.dot(h, wr_ref[...], preferred_element_type=_F32, precision=lax.Precision.HIGHEST) + b_ref[...]


def _relu_proj(parts, r, Wn, Wr, b):
    return pl.pallas_call(
        _relu_proj_body,
        grid=(N // RB,),
        in_specs=[
            pl.BlockSpec((RB, D_H), lambda i: (i, 0)),
            pl.BlockSpec((RB, D_H), lambda i: (i + N // RB, 0)),
            pl.BlockSpec((RB, D_H), lambda i: (i, 0)),
            pl.BlockSpec((D_H, D_H), lambda i: (0, 0)),
            pl.BlockSpec((D_H, D_H), lambda i: (0, 0)),
            pl.BlockSpec((1, D_H), lambda i: (0, 0)),
        ],
        out_specs=[
            pl.BlockSpec((RB, D_H), lambda i: (i, 0)),
            pl.BlockSpec((RB, D_H), lambda i: (i, 0)),
            pl.BlockSpec((RB, D_H), lambda i: (i, 0)),
        ],
        out_shape=[
            jax.ShapeDtypeStruct((N, D_H), _F32),
            jax.ShapeDtypeStruct((N, D_H), _F32),
            jax.ShapeDtypeStruct((N, D_H), _F32),
        ],
    )(parts, parts, r, Wn, Wr, b.reshape(1, D_H))


def _relu_body(p0_ref, p1_ref, r_ref, h_ref):
    h_ref[...] = jnp.maximum(p0_ref[...] + p1_ref[...] + r_ref[...], 0.0)


def _relu_combine(parts, r):
    return pl.pallas_call(
        _relu_body,
        grid=(N // RB,),
        in_specs=[
            pl.BlockSpec((RB, D_H), lambda i: (i, 0)),
            pl.BlockSpec((RB, D_H), lambda i: (i + N // RB, 0)),
            pl.BlockSpec((RB, D_H), lambda i: (i, 0)),
        ],
        out_specs=pl.BlockSpec((RB, D_H), lambda i: (i, 0)),
        out_shape=jax.ShapeDtypeStruct((N, D_H), _F32),
    )(parts, parts, r)


def _sc_pool(h_ref, off_ref, pool_ref, off_v, hbuf, pbuf, wid):
    pltpu.sync_copy(off_ref, off_v)
    neg_inf = jnp.full((16,), _NEG_INF, _F32)
    zero = jnp.zeros((16,), _F32)
    for rr in range(26, 32):
        for k in range(8):
            pbuf[rr, pl.ds(16 * k, 16)] = zero
    for j in range(13):
        g = wid * 13 + j
        ovec = off_v[pl.ds(g, 16)]
        start = ovec[0]
        cnt = ovec[1] - start
        astart = (start // 8) * 8
        boff = start - astart
        total = boff + cnt

        def chunk_body(ci, carry):
            acc = carry
            cb = astart + ci * CH
            cstart = jnp.minimum(cb, N - CH)
            base = cb - cstart
            pltpu.sync_copy(h_ref.at[pl.ds(cstart, CH)], hbuf)
            lo = jnp.maximum(boff - ci * CH, 0)
            hi = jnp.minimum(total - ci * CH, CH)

            def row_body(r2, acc2):
                rix = base + r2
                new = []
                for k in range(8):
                    v = hbuf[rix, pl.ds(16 * k, 16)]
                    new.append(jnp.maximum(acc2[k], v))
                for k in range(8):
                    v = hbuf[rix, pl.ds(16 * k, 16)]
                    new.append(acc2[8 + k] + v)
                return tuple(new)

            return lax.fori_loop(lo, hi, row_body, acc)

        init = tuple([neg_inf] * 8 + [zero] * 8)
        nch = jnp.where(cnt > 0, (total + CH - 1) // CH, 0)
        acc = lax.fori_loop(0, nch, chunk_body, init)
        nonempty = cnt > 0
        for k in range(8):
            pbuf[2 * j, pl.ds(16 * k, 16)] = jnp.where(nonempty, acc[k], zero)
            pbuf[2 * j + 1, pl.ds(16 * k, 16)] = acc[8 + k]
    pltpu.sync_copy(pbuf, pool_ref.at[pl.ds(32 * wid, 32)])


def _make_sc_kernel(do_scatter, do_pool):
    out_type = []
    if do_scatter:
        out_type.append(jax.ShapeDtypeStruct((NC * N, D_H), _F32))
    if do_pool:
        out_type.append(jax.ShapeDtypeStruct((32 * NW, D_H), _F32))

    scratch = []
    if do_scatter:
        scratch += [
            pltpu.VMEM((NWIN, WIN), jnp.int32),
            pltpu.VMEM((NWIN, WIN), jnp.int32),
            pltpu.VMEM((2, WIN, D_H), _F32),
            pltpu.VMEM_SHARED((N, D_H), _F32),
            pltpu.SemaphoreType.DMA,
            pltpu.SemaphoreType.DMA,
            pltpu.SemaphoreType.DMA,
        ]
    if do_pool:
        scratch += [
            pltpu.VMEM((512,), jnp.int32),
            pltpu.VMEM((CH, D_H), _F32),
            pltpu.VMEM((32, D_H), _F32),
        ]

    mesh = plsc.VectorSubcoreMesh(core_axis_name="c", subcore_axis_name="s")

    def body(*refs):
        refs = list(refs)
        m_ref = srcw_ref = dstw_ref = zeros_ref = None
        h_ref = off_ref = None
        if do_scatter:
            m_ref, srcw_ref, dstw_ref, zeros_ref = refs[:4]
            del refs[:4]
        if do_pool:
            h_ref, off_ref = refs[:2]
            del refs[:2]
        parts_ref = pool_ref = None
        if do_scatter:
            parts_ref = refs.pop(0)
        if do_pool:
            pool_ref = refs.pop(0)
        if do_scatter:
            idx_s, idx_d, rows, agg, zsem, gsem0, gsem1 = refs[:7]
            del refs[:7]
        if do_pool:
            off_v, hbuf, pbuf = refs[:3]
            del refs[:3]

        c = lax.axis_index("c")
        s = lax.axis_index("s")
        wid = s * NC + c

        if do_scatter:
            zcp = pltpu.async_copy(
                zeros_ref.at[pl.ds(s * SLAB, SLAB)],
                agg.at[pl.ds(s * SLAB, SLAB)], zsem)

            @pl.when(s == NS - 1)
            def _():
                pltpu.async_copy(
                    zeros_ref.at[pl.ds(NS * SLAB, TAIL)],
                    agg.at[pl.ds(NS * SLAB, TAIL)], zsem).wait()

            pltpu.sync_copy(srcw_ref.at[wid], idx_s)
            pltpu.sync_copy(dstw_ref.at[wid], idx_d)

        if do_pool:
            _sc_pool(h_ref, off_ref, pool_ref, off_v, hbuf, pbuf, wid)

        if do_scatter:
            zcp.wait()
            plsc.subcore_barrier()
            cp0 = pltpu.async_copy(m_ref.at[idx_s.at[0]], rows.at[0], gsem0)
            sems = (gsem0, gsem1)
            for w in range(NWIN):
                cur = w % 2
                if w + 1 < NWIN:
                    pltpu.async_copy(
                        m_ref.at[idx_s.at[w + 1]], rows.at[1 - cur],
                        sems[1 - cur])
                if w == 0:
                    cp0.wait()
                else:
                    pltpu.make_async_copy(
                        m_ref.at[idx_s.at[w]], rows.at[cur], sems[cur]).wait()
                pltpu.sync_copy(rows.at[cur], agg.at[idx_d.at[w]], add=True)
            plsc.subcore_barrier()
            pltpu.sync_copy(
                agg.at[pl.ds(s * SLAB, SLAB)],
                parts_ref.at[pl.ds(c * N + s * SLAB, SLAB)])

            @pl.when(s == NS - 1)
            def _():
                pltpu.sync_copy(
                    agg.at[pl.ds(NS * SLAB, TAIL)],
                    parts_ref.at[pl.ds(c * N + NS * SLAB, TAIL)])

    kfn = functools.partial(
        pl.kernel, out_type=out_type, mesh=mesh, scratch_types=scratch,
    )(body)
    return kfn


_sc_scatter = _make_sc_kernel(True, False)
_sc_scatter_pool = _make_sc_kernel(True, True)
_sc_pool_only = _make_sc_kernel(False, True)


def _head_body(t1_ref, t2_ref, t3_ref, mb_ref, ddi_ref, w1a_ref, w1b_ref,
               b1_ref, w2_ref, b2_ref, w3_ref, b3_ref, out_ref):
    tot = t1_ref[...] + t2_ref[...] + t3_ref[...]
    v = jnp.dot(mb_ref[...], tot, preferred_element_type=_F32, precision=lax.Precision.HIGHEST)
    z = jnp.dot(v, w1a_ref[...], preferred_element_type=_F32,
                precision=lax.Precision.HIGHEST) + jnp.dot(ddi_ref[...], w1b_ref[...],
                                   preferred_element_type=_F32, precision=lax.Precision.HIGHEST) + b1_ref[...]
    z = jnp.maximum(z, 0.0)
    z = jnp.maximum(jnp.dot(z, w2_ref[...], preferred_element_type=_F32,
                    precision=lax.Precision.HIGHEST) + b2_ref[...], 0.0)
    out_ref[...] = jnp.dot(z, w3_ref[...], preferred_element_type=_F32,
                           precision=lax.Precision.HIGHEST) + b3_ref[...]


def _head(p1, p2, p3, Mbig, DDI, lin1_W, lin1_b, lin2_W, lin2_b, lin3_W, lin3_b):
    return pl.pallas_call(
        _head_body,
        out_shape=jax.ShapeDtypeStruct((B, 1), _F32),
    )(p1, p2, p3, Mbig, DDI,
      lin1_W[:D_H], lin1_W[D_H:], lin1_b.reshape(1, -1),
      lin2_W, lin2_b.reshape(1, -1), lin3_W, lin3_b.reshape(1, -1))


def kernel(x, edge_index, batch, DDI_features, protein_mask,
           W1_root, W1_rel, b1, W2_root, W2_rel, b2, W3_root, W3_rel, b3,
           lin1_W, lin1_b, lin2_W, lin2_b, lin3_W, lin3_b):
    srcw = edge_index[0].reshape(NW, NWIN, WIN)
    dstw = edge_index[1].reshape(NW, NWIN, WIN)
    zeros = jnp.zeros((N, D_H), _F32)

    off = _compute_offsets(batch)

    bb = jnp.arange(B)[:, None]
    tt = 25 * bb + jnp.arange(P)[None, :]
    rowmap = 32 * (tt // 26) + tt % 26
    Mbig = jnp.zeros((B, 32 * NW), _F32).at[bb, rowmap].set(
        protein_mask.astype(_F32))

    m1, r1 = _proj(x, W1_rel, W1_root, b1)
    (parts1,) = _sc_scatter(m1, srcw, dstw, zeros)
    h1, m2, r2 = _relu_proj(parts1, r1, W2_rel, W2_root, b2)

    parts2, pool1 = _sc_scatter_pool(m2, srcw, dstw, zeros, h1, off)
    h2, m3, r3 = _relu_proj(parts2, r2, W3_rel, W3_root, b3)

    parts3, pool2 = _sc_scatter_pool(m3, srcw, dstw, zeros, h2, off)
    h3 = _relu_combine(parts3, r3)

    (pool3,) = _sc_pool_only(h3, off)

    return _head(pool1, pool2, pool3, Mbig, DDI_features,
                 lin1_W, lin1_b, lin2_W, lin2_b, lin3_W, lin3_b)


# speedup vs baseline: 6.9182x; 1.0025x over previous
import functools

import jax
import jax.numpy as jnp
from jax import lax
from jax.experimental import pallas as pl
from jax.experimental.pallas import tpu as pltpu
from jax.experimental.pallas import tpu_sc as plsc

N = 10000
E = 160000
G = 400
GP = 416
B = 32
P = 25
D_IN = 256
D_H = 128

NC = 2
NS = 16
NW = NC * NS
EPW = E // NW
WIN = 100
NWIN = EPW // WIN
CH = 32
SLAB = 624
TAIL = N - NS * SLAB
RB = 400

_F32 = jnp.float32
_NEG_INF = float("-inf")


def _offsets_body(batch_ref, off_ref):
    i = pl.program_id(0)
    b = batch_ref[...]
    j = lax.broadcasted_iota(jnp.int32, (RB, 512), 1)
    cnt = jnp.sum((b < j).astype(jnp.int32), axis=0, keepdims=True)

    @pl.when(i == 0)
    def _():
        off_ref[...] = jnp.zeros_like(off_ref)

    off_ref[...] += cnt


def _compute_offsets(batch):
    bcol = batch.reshape(N, 1)
    out = pl.pallas_call(
        _offsets_body,
        grid=(N // RB,),
        in_specs=[pl.BlockSpec((RB, 1), lambda i: (i, 0))],
        out_specs=pl.BlockSpec((1, 512), lambda i: (0, 0)),
        out_shape=jax.ShapeDtypeStruct((1, 512), jnp.int32),
    )(bcol)
    return out.reshape(512)


def _proj_body(h_ref, wn_ref, wr_ref, b_ref, m_ref, r_ref):
    h = h_ref[...]
    m_ref[...] = jnp.dot(h, wn_ref[...], preferred_element_type=_F32, precision=lax.Precision.HIGHEST)
    r_ref[...] = jnp.dot(h, wr_ref[...], preferred_element_type=_F32, precision=lax.Precision.HIGHEST) + b_ref[...]


def _proj(h, Wn, Wr, b):
    din = h.shape[1]
    return pl.pallas_call(
        _proj_body,
        grid=(N // RB,),
        in_specs=[
            pl.BlockSpec((RB, din), lambda i: (i, 0)),
            pl.BlockSpec((din, D_H), lambda i: (0, 0)),
            pl.BlockSpec((din, D_H), lambda i: (0, 0)),
            pl.BlockSpec((1, D_H), lambda i: (0, 0)),
        ],
        out_specs=[
            pl.BlockSpec((RB, D_H), lambda i: (i, 0)),
            pl.BlockSpec((RB, D_H), lambda i: (i, 0)),
        ],
        out_shape=[
            jax.ShapeDtypeStruct((N, D_H), _F32),
            jax.ShapeDtypeStruct((N, D_H), _F32),
        ],
    )(h, Wn, Wr, b.reshape(1, D_H))


def _relu_proj_body(p0_ref, p1_ref, r_ref, wn_ref, wr_ref, b_ref,
                    h_ref, m2_ref, r2_ref):
    h = jnp.maximum(p0_ref[...] + p1_ref[...] + r_ref[...], 0.0)
    h_ref[...] = h
    m2_ref[...] = jnp.dot(h, wn_ref[...], preferred_element_type=_F32, precision=lax.Precision.HIGHEST)
    r2_ref[...] = jnp.dot(h, wr_ref[...], preferred_element_type=_F32, precision=lax.Precision.HIGHEST) + b_ref[...]


def _relu_proj(parts, r, Wn, Wr, b):
    return pl.pallas_call(
        _relu_proj_body,
        grid=(N // RB,),
        in_specs=[
            pl.BlockSpec((RB, D_H), lambda i: (i, 0)),
            pl.BlockSpec((RB, D_H), lambda i: (i + N // RB, 0)),
            pl.BlockSpec((RB, D_H), lambda i: (i, 0)),
            pl.BlockSpec((D_H, D_H), lambda i: (0, 0)),
            pl.BlockSpec((D_H, D_H), lambda i: (0, 0)),
            pl.BlockSpec((1, D_H), lambda i: (0, 0)),
        ],
        out_specs=[
            pl.BlockSpec((RB, D_H), lambda i: (i, 0)),
            pl.BlockSpec((RB, D_H), lambda i: (i, 0)),
            pl.BlockSpec((RB, D_H), lambda i: (i, 0)),
        ],
        out_shape=[
            jax.ShapeDtypeStruct((N, D_H), _F32),
            jax.ShapeDtypeStruct((N, D_H), _F32),
            jax.ShapeDtypeStruct((N, D_H), _F32),
        ],
    )(parts, parts, r, Wn, Wr, b.reshape(1, D_H))


def _relu_body(p0_ref, p1_ref, r_ref, h_ref):
    h_ref[...] = jnp.maximum(p0_ref[...] + p1_ref[...] + r_ref[...], 0.0)


def _relu_combine(parts, r):
    return pl.pallas_call(
        _relu_body,
        grid=(N // RB,),
        in_specs=[
            pl.BlockSpec((RB, D_H), lambda i: (i, 0)),
            pl.BlockSpec((RB, D_H), lambda i: (i + N // RB, 0)),
            pl.BlockSpec((RB, D_H), lambda i: (i, 0)),
        ],
        out_specs=pl.BlockSpec((RB, D_H), lambda i: (i, 0)),
        out_shape=jax.ShapeDtypeStruct((N, D_H), _F32),
    )(parts, parts, r)


def _sc_pool(h_ref, off_ref, pool_ref, off_v, hbuf, pbuf, wid):
    pltpu.sync_copy(off_ref, off_v)
    neg_inf = jnp.full((16,), _NEG_INF, _F32)
    zero = jnp.zeros((16,), _F32)
    for rr in range(26, 32):
        for k in range(8):
            pbuf[rr, pl.ds(16 * k, 16)] = zero
    for j in range(13):
        g = wid * 13 + j
        ovec = off_v[pl.ds(g, 16)]
        start = ovec[0]
        cnt = ovec[1] - start
        astart = (start // 8) * 8
        boff = start - astart
        total = boff + cnt

        def chunk_body(ci, carry):
            acc = carry
            cb = astart + ci * CH
            cstart = jnp.minimum(cb, N - CH)
            base = cb - cstart
            pltpu.sync_copy(h_ref.at[pl.ds(cstart, CH)], hbuf)
            lo = jnp.maximum(boff - ci * CH, 0)
            hi = jnp.minimum(total - ci * CH, CH)

            def row_body(r2, acc2):
                rix = base + r2
                new = []
                for k in range(8):
                    v = hbuf[rix, pl.ds(16 * k, 16)]
                    new.append(jnp.maximum(acc2[k], v))
                for k in range(8):
                    v = hbuf[rix, pl.ds(16 * k, 16)]
                    new.append(acc2[8 + k] + v)
                return tuple(new)

            return lax.fori_loop(lo, hi, row_body, acc)

        init = tuple([neg_inf] * 8 + [zero] * 8)
        nch = jnp.where(cnt > 0, (total + CH - 1) // CH, 0)
        acc = lax.fori_loop(0, nch, chunk_body, init)
        nonempty = cnt > 0
        for k in range(8):
            pbuf[2 * j, pl.ds(16 * k, 16)] = jnp.where(nonempty, acc[k], zero)
            pbuf[2 * j + 1, pl.ds(16 * k, 16)] = acc[8 + k]
    pltpu.sync_copy(pbuf, pool_ref.at[pl.ds(32 * wid, 32)])


def _make_sc_kernel(do_scatter, do_pool):
    out_type = []
    if do_scatter:
        out_type.append(jax.ShapeDtypeStruct((NC * N, D_H), _F32))
    if do_pool:
        out_type.append(jax.ShapeDtypeStruct((32 * NW, D_H), _F32))

    scratch = []
    if do_scatter:
        scratch += [
            pltpu.VMEM((NWIN, WIN), jnp.int32),
            pltpu.VMEM((NWIN, WIN), jnp.int32),
            pltpu.VMEM((2, WIN, D_H), _F32),
            pltpu.VMEM_SHARED((N, D_H), _F32),
            pltpu.SemaphoreType.DMA,
            pltpu.SemaphoreType.DMA,
            pltpu.SemaphoreType.DMA,
            pltpu.SemaphoreType.DMA,
            pltpu.SemaphoreType.DMA,
        ]
    if do_pool:
        scratch += [
            pltpu.VMEM((512,), jnp.int32),
            pltpu.VMEM((CH, D_H), _F32),
            pltpu.VMEM((32, D_H), _F32),
        ]

    mesh = plsc.VectorSubcoreMesh(core_axis_name="c", subcore_axis_name="s")

    def body(*refs):
        refs = list(refs)
        m_ref = srcw_ref = dstw_ref = zeros_ref = None
        h_ref = off_ref = None
        if do_scatter:
            m_ref, srcw_ref, dstw_ref, zeros_ref = refs[:4]
            del refs[:4]
        if do_pool:
            h_ref, off_ref = refs[:2]
            del refs[:2]
        parts_ref = pool_ref = None
        if do_scatter:
            parts_ref = refs.pop(0)
        if do_pool:
            pool_ref = refs.pop(0)
        if do_scatter:
            idx_s, idx_d, rows, agg, zsem, gsem0, gsem1, ssem0, ssem1 = refs[:9]
            del refs[:9]
        if do_pool:
            off_v, hbuf, pbuf = refs[:3]
            del refs[:3]

        c = lax.axis_index("c")
        s = lax.axis_index("s")
        wid = s * NC + c

        if do_scatter:
            zcp = pltpu.async_copy(
                zeros_ref.at[pl.ds(s * SLAB, SLAB)],
                agg.at[pl.ds(s * SLAB, SLAB)], zsem)

            @pl.when(s == NS - 1)
            def _():
                pltpu.async_copy(
                    zeros_ref.at[pl.ds(NS * SLAB, TAIL)],
                    agg.at[pl.ds(NS * SLAB, TAIL)], zsem).wait()

            pltpu.sync_copy(srcw_ref.at[wid], idx_s)
            pltpu.sync_copy(dstw_ref.at[wid], idx_d)

        if do_pool:
            _sc_pool(h_ref, off_ref, pool_ref, off_v, hbuf, pbuf, wid)

        if do_scatter:
            zcp.wait()
            plsc.subcore_barrier()
            gsems = (gsem0, gsem1)
            ssems = (ssem0, ssem1)
            gd = [None, None]
            sd = [None, None]
            gd[0] = pltpu.async_copy(m_ref.at[idx_s.at[0]], rows.at[0], gsem0)
            for w in range(NWIN):
                cur = w % 2
                if w + 1 < NWIN:
                    if sd[1 - cur] is not None:
                        sd[1 - cur].wait()
                    gd[1 - cur] = pltpu.async_copy(
                        m_ref.at[idx_s.at[w + 1]], rows.at[1 - cur],
                        gsems[1 - cur])
                gd[cur].wait()
                sd[cur] = pltpu.async_copy(
                    rows.at[cur], agg.at[idx_d.at[w]], ssems[cur], add=True)
            sd[(NWIN - 1) % 2].wait()
            sd[NWIN % 2].wait()
            plsc.subcore_barrier()
            pltpu.sync_copy(
                agg.at[pl.ds(s * SLAB, SLAB)],
                parts_ref.at[pl.ds(c * N + s * SLAB, SLAB)])

            @pl.when(s == NS - 1)
            def _():
                pltpu.sync_copy(
                    agg.at[pl.ds(NS * SLAB, TAIL)],
                    parts_ref.at[pl.ds(c * N + NS * SLAB, TAIL)])

    kfn = functools.partial(
        pl.kernel, out_type=out_type, mesh=mesh, scratch_types=scratch,
    )(body)
    return kfn


_sc_scatter = _make_sc_kernel(True, False)
_sc_scatter_pool = _make_sc_kernel(True, True)
_sc_pool_only = _make_sc_kernel(False, True)


def _head_body(t1_ref, t2_ref, t3_ref, mb_ref, ddi_ref, w1a_ref, w1b_ref,
               b1_ref, w2_ref, b2_ref, w3_ref, b3_ref, out_ref):
    tot = t1_ref[...] + t2_ref[...] + t3_ref[...]
    v = jnp.dot(mb_ref[...], tot, preferred_element_type=_F32, precision=lax.Precision.HIGHEST)
    z = jnp.dot(v, w1a_ref[...], preferred_element_type=_F32,
                precision=lax.Precision.HIGHEST) + jnp.dot(ddi_ref[...], w1b_ref[...],
                                   preferred_element_type=_F32, precision=lax.Precision.HIGHEST) + b1_ref[...]
    z = jnp.maximum(z, 0.0)
    z = jnp.maximum(jnp.dot(z, w2_ref[...], preferred_element_type=_F32,
                    precision=lax.Precision.HIGHEST) + b2_ref[...], 0.0)
    out_ref[...] = jnp.dot(z, w3_ref[...], preferred_element_type=_F32,
                           precision=lax.Precision.HIGHEST) + b3_ref[...]


def _head(p1, p2, p3, Mbig, DDI, lin1_W, lin1_b, lin2_W, lin2_b, lin3_W, lin3_b):
    return pl.pallas_call(
        _head_body,
        out_shape=jax.ShapeDtypeStruct((B, 1), _F32),
    )(p1, p2, p3, Mbig, DDI,
      lin1_W[:D_H], lin1_W[D_H:], lin1_b.reshape(1, -1),
      lin2_W, lin2_b.reshape(1, -1), lin3_W, lin3_b.reshape(1, -1))


def kernel(x, edge_index, batch, DDI_features, protein_mask,
           W1_root, W1_rel, b1, W2_root, W2_rel, b2, W3_root, W3_rel, b3,
           lin1_W, lin1_b, lin2_W, lin2_b, lin3_W, lin3_b):
    srcw = edge_index[0].reshape(NW, NWIN, WIN)
    dstw = edge_index[1].reshape(NW, NWIN, WIN)
    zeros = jnp.zeros((N, D_H), _F32)

    off = _compute_offsets(batch)

    bb = jnp.arange(B)[:, None]
    tt = 25 * bb + jnp.arange(P)[None, :]
    rowmap = 32 * (tt // 26) + tt % 26
    Mbig = jnp.zeros((B, 32 * NW), _F32).at[bb, rowmap].set(
        protein_mask.astype(_F32))

    m1, r1 = _proj(x, W1_rel, W1_root, b1)
    (parts1,) = _sc_scatter(m1, srcw, dstw, zeros)
    h1, m2, r2 = _relu_proj(parts1, r1, W2_rel, W2_root, b2)

    parts2, pool1 = _sc_scatter_pool(m2, srcw, dstw, zeros, h1, off)
    h2, m3, r3 = _relu_proj(parts2, r2, W3_rel, W3_root, b3)

    parts3, pool2 = _sc_scatter_pool(m3, srcw, dstw, zeros, h2, off)
    h3 = _relu_combine(parts3, r3)

    (pool3,) = _sc_pool_only(h3, off)

    return _head(pool1, pool2, pool3, Mbig, DDI_features,
                 lin1_W, lin1_b, lin2_W, lin2_b, lin3_W, lin3_b)
